# baseline (speedup 1.0000x reference)
import jax
import jax.numpy as jnp
from jax.experimental import pallas as pl
from jax.experimental.pallas import tpu as pltpu


def _two_hop_body(a_ref, f_ref, w0_ref, w1_ref, o_ref):
    a = a_ref[...].astype(jnp.bfloat16)
    f = f_ref[...].astype(jnp.bfloat16)
    w0 = w0_ref[...]
    w1 = w1_ref[...]

    g = jnp.dot(f, w0, preferred_element_type=jnp.float32)
    h = jnp.dot(a, g.astype(jnp.bfloat16), preferred_element_type=jnp.float32)
    h = jnp.maximum(h, 0.0).astype(jnp.bfloat16)

    g = jnp.dot(h, w1, preferred_element_type=jnp.float32)
    h = jnp.dot(a, g.astype(jnp.bfloat16), preferred_element_type=jnp.float32)
    o_ref[...] = jnp.maximum(h, 0.0)


def kernel(a_norm, f_norm, w0, w1):
    d = w0.shape[-1]
    w0b = w0.astype(jnp.bfloat16)
    w1b = w1.astype(jnp.bfloat16)

    if a_norm.ndim == 2:
        n = a_norm.shape[0]
        vmem = pl.BlockSpec(memory_space=pltpu.MemorySpace.VMEM)
        return pl.pallas_call(
            _two_hop_body,
            out_shape=jax.ShapeDtypeStruct((n, d), jnp.float32),
            in_specs=[vmem, vmem, vmem, vmem],
            out_specs=vmem,
        )(a_norm, f_norm, w0b, w1b)

    b, n, _ = a_norm.shape
    return pl.pallas_call(
        _two_hop_body,
        out_shape=jax.ShapeDtypeStruct((b, n, d), jnp.float32),
        grid=(b,),
        in_specs=[
            pl.BlockSpec((pl.Squeezed(), n, n), lambda i: (i, 0, 0)),
            pl.BlockSpec((pl.Squeezed(), n, d), lambda i: (i, 0, 0)),
            pl.BlockSpec((d, d), lambda i: (0, 0)),
            pl.BlockSpec((d, d), lambda i: (0, 0)),
        ],
        out_specs=pl.BlockSpec((pl.Squeezed(), n, d), lambda i: (i, 0, 0)),
        compiler_params=pltpu.CompilerParams(
            dimension_semantics=("parallel",)),
    )(a_norm, f_norm, w0b, w1b)


# speedup vs baseline: 1.0651x; 1.0651x over previous
import jax
import jax.numpy as jnp
from jax.experimental import pallas as pl
from jax.experimental.pallas import tpu as pltpu


def _two_hop_body(a_ref, f_ref, w0_ref, w1_ref, o_ref):
    a = a_ref[...].astype(jnp.bfloat16)
    f = f_ref[...].astype(jnp.bfloat16)
    w0 = w0_ref[...].astype(jnp.bfloat16)
    w1 = w1_ref[...].astype(jnp.bfloat16)

    g = jnp.dot(f, w0, preferred_element_type=jnp.float32)
    h = jnp.dot(a, g.astype(jnp.bfloat16), preferred_element_type=jnp.float32)
    h = jnp.maximum(h, 0.0).astype(jnp.bfloat16)

    g = jnp.dot(h, w1, preferred_element_type=jnp.float32)
    h = jnp.dot(a, g.astype(jnp.bfloat16), preferred_element_type=jnp.float32)
    o_ref[...] = jnp.maximum(h, 0.0)


def kernel(a_norm, f_norm, w0, w1):
    d = w0.shape[-1]

    if a_norm.ndim == 2:
        n = a_norm.shape[0]
        vmem = pl.BlockSpec(memory_space=pltpu.MemorySpace.VMEM)
        return pl.pallas_call(
            _two_hop_body,
            out_shape=jax.ShapeDtypeStruct((n, d), jnp.float32),
            in_specs=[vmem, vmem, vmem, vmem],
            out_specs=vmem,
        )(a_norm, f_norm, w0, w1)

    b, n, _ = a_norm.shape
    return pl.pallas_call(
        _two_hop_body,
        out_shape=jax.ShapeDtypeStruct((b, n, d), jnp.float32),
        grid=(b,),
        in_specs=[
            pl.BlockSpec((pl.Squeezed(), n, n), lambda i: (i, 0, 0)),
            pl.BlockSpec((pl.Squeezed(), n, d), lambda i: (i, 0, 0)),
            pl.BlockSpec((d, d), lambda i: (0, 0)),
            pl.BlockSpec((d, d), lambda i: (0, 0)),
        ],
        out_specs=pl.BlockSpec((pl.Squeezed(), n, d), lambda i: (i, 0, 0)),
        compiler_params=pltpu.CompilerParams(
            dimension_semantics=("parallel",)),
    )(a_norm, f_norm, w0, w1)


# speedup vs baseline: 2.0827x; 1.9554x over previous
import jax
import jax.numpy as jnp
from jax.experimental import pallas as pl
from jax.experimental.pallas import tpu as pltpu

_GBLK = 4


def _two_hop_body(a_ref, f_ref, w0_ref, w1_ref, o_ref):
    g_blk, n, d = f_ref.shape
    w0 = w0_ref[...].astype(jnp.bfloat16)
    w1 = w1_ref[...].astype(jnp.bfloat16)
    f = f_ref[...].astype(jnp.bfloat16).reshape(g_blk * n, d)

    g0 = jnp.dot(f, w0, preferred_element_type=jnp.float32)
    g0 = g0.astype(jnp.bfloat16).reshape(g_blk, n, d)

    a = [a_ref[g].astype(jnp.bfloat16) for g in range(g_blk)]
    h1 = [
        jnp.maximum(
            jnp.dot(a[g], g0[g], preferred_element_type=jnp.float32), 0.0
        ).astype(jnp.bfloat16)
        for g in range(g_blk)
    ]

    h1_flat = jnp.concatenate(h1, axis=0)
    g1 = jnp.dot(h1_flat, w1, preferred_element_type=jnp.float32)
    g1 = g1.astype(jnp.bfloat16).reshape(g_blk, n, d)

    for g in range(g_blk):
        o_ref[g] = jnp.maximum(
            jnp.dot(a[g], g1[g], preferred_element_type=jnp.float32), 0.0)


def _single_graph_body(a_ref, f_ref, w0_ref, w1_ref, o_ref):
    a = a_ref[...].astype(jnp.bfloat16)
    f = f_ref[...].astype(jnp.bfloat16)
    w0 = w0_ref[...].astype(jnp.bfloat16)
    w1 = w1_ref[...].astype(jnp.bfloat16)
    g = jnp.dot(f, w0, preferred_element_type=jnp.float32)
    h = jnp.dot(a, g.astype(jnp.bfloat16), preferred_element_type=jnp.float32)
    h = jnp.maximum(h, 0.0).astype(jnp.bfloat16)
    g = jnp.dot(h, w1, preferred_element_type=jnp.float32)
    h = jnp.dot(a, g.astype(jnp.bfloat16), preferred_element_type=jnp.float32)
    o_ref[...] = jnp.maximum(h, 0.0)


def kernel(a_norm, f_norm, w0, w1):
    d = w0.shape[-1]

    if a_norm.ndim == 2:
        n = a_norm.shape[0]
        vmem = pl.BlockSpec(memory_space=pltpu.MemorySpace.VMEM)
        return pl.pallas_call(
            _single_graph_body,
            out_shape=jax.ShapeDtypeStruct((n, d), jnp.float32),
            in_specs=[vmem, vmem, vmem, vmem],
            out_specs=vmem,
        )(a_norm, f_norm, w0, w1)

    b, n, _ = a_norm.shape
    gblk = _GBLK if b % _GBLK == 0 else 1
    return pl.pallas_call(
        _two_hop_body,
        out_shape=jax.ShapeDtypeStruct((b, n, d), jnp.float32),
        grid=(b // gblk,),
        in_specs=[
            pl.BlockSpec((gblk, n, n), lambda i: (i, 0, 0)),
            pl.BlockSpec((gblk, n, d), lambda i: (i, 0, 0)),
            pl.BlockSpec((d, d), lambda i: (0, 0)),
            pl.BlockSpec((d, d), lambda i: (0, 0)),
        ],
        out_specs=pl.BlockSpec((gblk, n, d), lambda i: (i, 0, 0)),
        compiler_params=pltpu.CompilerParams(
            dimension_semantics=("parallel",)),
    )(a_norm, f_norm, w0, w1)


# speedup vs baseline: 2.2408x; 1.0759x over previous
import jax
import jax.numpy as jnp
from jax.experimental import pallas as pl
from jax.experimental.pallas import tpu as pltpu

_GBLK = 8


def _two_hop_body(a_ref, f_ref, w0_ref, w1_ref, o_ref):
    g_blk, n, d = f_ref.shape
    w0 = w0_ref[...].astype(jnp.bfloat16)
    w1 = w1_ref[...].astype(jnp.bfloat16)
    f = f_ref[...].astype(jnp.bfloat16).reshape(g_blk * n, d)

    g0 = jnp.dot(f, w0, preferred_element_type=jnp.float32)
    g0 = g0.astype(jnp.bfloat16).reshape(g_blk, n, d)

    a = [a_ref[g].astype(jnp.bfloat16) for g in range(g_blk)]
    h1 = [
        jnp.maximum(
            jnp.dot(a[g], g0[g], preferred_element_type=jnp.float32), 0.0
        ).astype(jnp.bfloat16)
        for g in range(g_blk)
    ]

    h1_flat = jnp.concatenate(h1, axis=0)
    g1 = jnp.dot(h1_flat, w1, preferred_element_type=jnp.float32)
    g1 = g1.astype(jnp.bfloat16).reshape(g_blk, n, d)

    for g in range(g_blk):
        o_ref[g] = jnp.maximum(
            jnp.dot(a[g], g1[g], preferred_element_type=jnp.float32), 0.0)


def _single_graph_body(a_ref, f_ref, w0_ref, w1_ref, o_ref):
    a = a_ref[...].astype(jnp.bfloat16)
    f = f_ref[...].astype(jnp.bfloat16)
    w0 = w0_ref[...].astype(jnp.bfloat16)
    w1 = w1_ref[...].astype(jnp.bfloat16)
    g = jnp.dot(f, w0, preferred_element_type=jnp.float32)
    h = jnp.dot(a, g.astype(jnp.bfloat16), preferred_element_type=jnp.float32)
    h = jnp.maximum(h, 0.0).astype(jnp.bfloat16)
    g = jnp.dot(h, w1, preferred_element_type=jnp.float32)
    h = jnp.dot(a, g.astype(jnp.bfloat16), preferred_element_type=jnp.float32)
    o_ref[...] = jnp.maximum(h, 0.0)


def kernel(a_norm, f_norm, w0, w1):
    d = w0.shape[-1]

    if a_norm.ndim == 2:
        n = a_norm.shape[0]
        vmem = pl.BlockSpec(memory_space=pltpu.MemorySpace.VMEM)
        return pl.pallas_call(
            _single_graph_body,
            out_shape=jax.ShapeDtypeStruct((n, d), jnp.float32),
            in_specs=[vmem, vmem, vmem, vmem],
            out_specs=vmem,
        )(a_norm, f_norm, w0, w1)

    b, n, _ = a_norm.shape
    gblk = _GBLK if b % _GBLK == 0 else 1
    return pl.pallas_call(
        _two_hop_body,
        out_shape=jax.ShapeDtypeStruct((b, n, d), jnp.float32),
        grid=(b // gblk,),
        in_specs=[
            pl.BlockSpec((gblk, n, n), lambda i: (i, 0, 0)),
            pl.BlockSpec((gblk, n, d), lambda i: (i, 0, 0)),
            pl.BlockSpec((d, d), lambda i: (0, 0)),
            pl.BlockSpec((d, d), lambda i: (0, 0)),
        ],
        out_specs=pl.BlockSpec((gblk, n, d), lambda i: (i, 0, 0)),
        compiler_params=pltpu.CompilerParams(
            dimension_semantics=("parallel",)),
    )(a_norm, f_norm, w0, w1)
